# baseline (speedup 1.0000x reference)
import jax
import jax.numpy as jnp
from jax import lax
from jax.experimental import pallas as pl
from jax.experimental.pallas import tpu as pltpu

B = 64
HW = 224
OHW = 112
S = OHW * OHW
C_IN = 3
E = 8
K = 2
N_CLASSES = 1000
G_CH = 16
E_CH = 32
KTAPS = 27
KP = KTAPS + 1
IPB = 2


def _im2col_router_body(x_ref, l_ref, r_ref, wg_ref, wl_ref, blin_ref,
                        p_ref, probs_ref, idx_ref, pw_ref, aux_ref, hg_scr):
    i = pl.program_id(0)
    for j in range(IPB):
        rs = []
        for c in range(C_IN):
            xc = x_ref[j, c].astype(jnp.bfloat16)
            row = []
            for ky in range(3):
                row.append(jnp.dot(l_ref[ky], xc,
                                   preferred_element_type=jnp.float32)
                           .astype(jnp.bfloat16))
            rs.append(row)
        pieces = []
        for c in range(C_IN):
            for ky in range(3):
                for kx in range(3):
                    pieces.append(jnp.dot(rs[c][ky], r_ref[kx],
                                          preferred_element_type=jnp.float32)
                                  .astype(jnp.bfloat16))
        pieces.append(jnp.ones((OHW, OHW), jnp.bfloat16))
        p28 = jnp.stack(pieces, axis=0).reshape(KP, S)
        p_ref[j] = p28

        hg = jnp.dot(wg_ref[...], p28, preferred_element_type=jnp.float32)
        hg = jax.nn.relu(hg)
        pooled = jnp.sum(hg, axis=1, keepdims=True) / S
        hg_scr[pl.ds(i * IPB + j, 1), :] = pooled.reshape(1, G_CH)

    @pl.when(i == B // IPB - 1)
    def _routing():
        hgt = hg_scr[...].T
        logits = jnp.dot(wl_ref[...], hgt, preferred_element_type=jnp.float32)
        logits = logits + blin_ref[...]
        m = jnp.max(logits, axis=0, keepdims=True)
        ex = jnp.exp(logits - m)
        probs = ex / jnp.sum(ex, axis=0, keepdims=True)
        probs_ref[...] = probs.T
        iota = lax.broadcasted_iota(jnp.int32, (E, B), 0)
        p1 = jnp.max(probs, axis=0, keepdims=True)
        i1 = jnp.min(jnp.where(probs == p1, iota, E), axis=0, keepdims=True)
        masked = jnp.where(iota == i1, -1.0, probs)
        p2 = jnp.max(masked, axis=0, keepdims=True)
        i2 = jnp.min(jnp.where(masked == p2, iota, E), axis=0, keepdims=True)
        idx_ref[...] = jnp.concatenate([i1, i2], axis=0)
        pw_ref[...] = jnp.concatenate([p1, p2], axis=0)
        mp = jnp.mean(probs, axis=1, keepdims=True)
        d = mp - (1.0 / E)
        aux_ref[...] = jnp.mean(d * d, keepdims=True).reshape(1, 1)


def _expert_body(idx_ref, pw_ref, p_ref, wa0_ref, wa1_ref, la0_ref, la1_ref,
                 bla0_ref, bla1_ref, wb0_ref, wb1_ref, lb0_ref, lb1_ref,
                 blb0_ref, blb1_ref, out_ref):
    i = pl.program_id(0)
    wrefs = ((wa0_ref, wa1_ref, la0_ref, la1_ref, bla0_ref, bla1_ref),
             (wb0_ref, wb1_ref, lb0_ref, lb1_ref, blb0_ref, blb1_ref))
    for j in range(IPB):
        w0_ref, w1_ref, l0_ref, l1_ref, bl0_ref, bl1_ref = wrefs[j]
        p = p_ref[j]
        p0 = pw_ref[0, i * IPB + j]
        p1 = pw_ref[1, i * IPB + j]

        w = jnp.concatenate([w0_ref[0], w1_ref[0]], axis=0)
        h = jnp.dot(w, p, preferred_element_type=jnp.float32)
        h = jax.nn.relu(h)
        mcol = jnp.sum(h, axis=1, keepdims=True) / S
        scale = jnp.concatenate(
            [jnp.full((E_CH, 1), p0, jnp.float32), jnp.full((E_CH, 1), p1, jnp.float32)],
            axis=0)
        mrow = (mcol * scale).reshape(1, 2 * E_CH)
        lcat = jnp.concatenate([l0_ref[0], l1_ref[0]], axis=0)
        o = jnp.dot(mrow, lcat, preferred_element_type=jnp.float32)
        o = o + p0 * bl0_ref[0] + p1 * bl1_ref[0]
        out_ref[j] = o


@jax.jit
def kernel(x, Wg_conv, bg_conv, Wg_lin, bg_lin, We_conv, be_conv, We_lin, be_lin):
    oidx = 2 * jnp.arange(OHW)
    iidx = jnp.arange(HW)
    lsel = jnp.stack(
        [(iidx[None, :] == (oidx + ky)[:, None]).astype(jnp.bfloat16)
         for ky in range(3)], axis=0)
    rsel = jnp.stack(
        [(iidx[:, None] == (oidx + kx)[None, :]).astype(jnp.bfloat16)
         for kx in range(3)], axis=0)
    wg = jnp.concatenate(
        [Wg_conv.reshape(G_CH, KTAPS), bg_conv[:, None]],
        axis=1).astype(jnp.bfloat16)

    p, probs, idx, pw, aux = pl.pallas_call(
        _im2col_router_body,
        grid=(B // IPB,),
        in_specs=[
            pl.BlockSpec((IPB, C_IN, HW, HW), lambda b: (b, 0, 0, 0)),
            pl.BlockSpec((3, OHW, HW), lambda b: (0, 0, 0)),
            pl.BlockSpec((3, HW, OHW), lambda b: (0, 0, 0)),
            pl.BlockSpec((G_CH, KP), lambda b: (0, 0)),
            pl.BlockSpec((E, G_CH), lambda b: (0, 0)),
            pl.BlockSpec((E, 1), lambda b: (0, 0)),
        ],
        out_specs=(
            pl.BlockSpec((IPB, KP, S), lambda b: (b, 0, 0)),
            pl.BlockSpec((B, E), lambda b: (0, 0)),
            pl.BlockSpec((K, B), lambda b: (0, 0)),
            pl.BlockSpec((K, B), lambda b: (0, 0)),
            pl.BlockSpec((1, 1), lambda b: (0, 0)),
        ),
        out_shape=(
            jax.ShapeDtypeStruct((B, KP, S), jnp.bfloat16),
            jax.ShapeDtypeStruct((B, E), jnp.float32),
            jax.ShapeDtypeStruct((K, B), jnp.int32),
            jax.ShapeDtypeStruct((K, B), jnp.float32),
            jax.ShapeDtypeStruct((1, 1), jnp.float32),
        ),
        scratch_shapes=[pltpu.VMEM((B, G_CH), jnp.float32)],
    )(x, lsel, rsel, wg, Wg_lin.T, bg_lin.reshape(E, 1))

    we = jnp.concatenate(
        [We_conv.reshape(E, E_CH, KTAPS), be_conv[:, :, None]],
        axis=2).astype(jnp.bfloat16)
    wl = We_lin
    bl = be_lin.reshape(E, 1, N_CLASSES)

    grid_spec = pltpu.PrefetchScalarGridSpec(
        num_scalar_prefetch=2,
        grid=(B // IPB,),
        in_specs=[
            pl.BlockSpec((IPB, KP, S), lambda i, idx_r, pw_r: (i, 0, 0)),
            pl.BlockSpec((1, E_CH, KP), lambda i, idx_r, pw_r: (idx_r[0, IPB * i], 0, 0)),
            pl.BlockSpec((1, E_CH, KP), lambda i, idx_r, pw_r: (idx_r[1, IPB * i], 0, 0)),
            pl.BlockSpec((1, E_CH, N_CLASSES), lambda i, idx_r, pw_r: (idx_r[0, IPB * i], 0, 0)),
            pl.BlockSpec((1, E_CH, N_CLASSES), lambda i, idx_r, pw_r: (idx_r[1, IPB * i], 0, 0)),
            pl.BlockSpec((1, 1, N_CLASSES), lambda i, idx_r, pw_r: (idx_r[0, IPB * i], 0, 0)),
            pl.BlockSpec((1, 1, N_CLASSES), lambda i, idx_r, pw_r: (idx_r[1, IPB * i], 0, 0)),
            pl.BlockSpec((1, E_CH, KP), lambda i, idx_r, pw_r: (idx_r[0, IPB * i + 1], 0, 0)),
            pl.BlockSpec((1, E_CH, KP), lambda i, idx_r, pw_r: (idx_r[1, IPB * i + 1], 0, 0)),
            pl.BlockSpec((1, E_CH, N_CLASSES), lambda i, idx_r, pw_r: (idx_r[0, IPB * i + 1], 0, 0)),
            pl.BlockSpec((1, E_CH, N_CLASSES), lambda i, idx_r, pw_r: (idx_r[1, IPB * i + 1], 0, 0)),
            pl.BlockSpec((1, 1, N_CLASSES), lambda i, idx_r, pw_r: (idx_r[0, IPB * i + 1], 0, 0)),
            pl.BlockSpec((1, 1, N_CLASSES), lambda i, idx_r, pw_r: (idx_r[1, IPB * i + 1], 0, 0)),
        ],
        out_specs=pl.BlockSpec((IPB, 1, N_CLASSES), lambda i, idx_r, pw_r: (i, 0, 0)),
    )
    final = pl.pallas_call(
        _expert_body,
        grid_spec=grid_spec,
        out_shape=jax.ShapeDtypeStruct((B, 1, N_CLASSES), jnp.float32),
    )(idx, pw, p, we, we, wl, wl, bl, bl, we, we, wl, wl, bl, bl)
    final = final.reshape(B, N_CLASSES)

    return final, probs, aux.reshape(())
